# baseline (speedup 1.0000x reference)
import functools

import jax
import jax.numpy as jnp
from jax import lax
from jax.experimental import pallas as pl
from jax.experimental.pallas import tpu as pltpu
from jax.experimental.pallas import tpu_sc as plsc

N = 10000
E = 320000
F_IN = 128
H = 64

NC = 2
NS = 16
NW = NC * NS
CHUNK = 128
CPW = 80
E_PAD = NW * CPW * CHUNK
TROWS = 632
N_SH = NS * TROWS

_mesh = plsc.VectorSubcoreMesh(core_axis_name="c", subcore_axis_name="s")
_sc_params = pltpu.CompilerParams(use_tc_tiling_on_sc=False)


@functools.partial(
    pl.kernel,
    out_type=jax.ShapeDtypeStruct((NC, N_SH, 16), jnp.float32),
    mesh=_mesh,
    scratch_types=[
        pltpu.VMEM((CPW, CHUNK), jnp.int32),
        pltpu.VMEM((CHUNK, 16), jnp.float32),
        pltpu.VMEM_SHARED((N_SH, 16), jnp.float32),
    ],
    compiler_params=_sc_params,
)
def _sc_degree(dst_hbm, ones_hbm, zero_hbm, out_hbm, dst_v, ones_v, acc_sh):
    cid = lax.axis_index("c")
    sid = lax.axis_index("s")
    wid = sid * NC + cid
    pltpu.sync_copy(zero_hbm, acc_sh.at[pl.ds(sid * TROWS, TROWS)])
    pltpu.sync_copy(dst_hbm.at[pl.ds(wid * CPW, CPW)], dst_v)
    pltpu.sync_copy(ones_hbm, ones_v)
    plsc.subcore_barrier()

    def body(j, carry):
        pltpu.sync_copy(ones_v, acc_sh.at[dst_v.at[j]], add=True)
        return carry

    lax.fori_loop(0, CPW, body, 0, unroll=4)
    plsc.subcore_barrier()
    pltpu.sync_copy(acc_sh.at[pl.ds(sid * TROWS, TROWS)],
                    out_hbm.at[cid, pl.ds(sid * TROWS, TROWS)])


@functools.partial(
    pl.kernel,
    out_type=jax.ShapeDtypeStruct((NC, N_SH, H), jnp.float32),
    mesh=_mesh,
    scratch_types=[
        pltpu.VMEM((CPW, CHUNK), jnp.int32),
        pltpu.VMEM((CPW, CHUNK), jnp.int32),
        pltpu.VMEM((4, CHUNK, H), jnp.float32),
        pltpu.VMEM_SHARED((N_SH, H), jnp.float32),
        pltpu.SemaphoreType.DMA,
        pltpu.SemaphoreType.DMA,
        pltpu.SemaphoreType.DMA,
        pltpu.SemaphoreType.DMA,
    ],
    compiler_params=_sc_params,
)
def _sc_aggregate(g_hbm, src_hbm, dst_hbm, zero_hbm, out_hbm,
                  src_v, dst_v, bufs, acc_sh, s0, s1, s2, s3):
    cid = lax.axis_index("c")
    sid = lax.axis_index("s")
    wid = sid * NC + cid
    sems = (s0, s1, s2, s3)
    pltpu.sync_copy(zero_hbm, acc_sh.at[pl.ds(sid * TROWS, TROWS)])
    pltpu.sync_copy(src_hbm.at[pl.ds(wid * CPW, CPW)], src_v)
    pltpu.sync_copy(dst_hbm.at[pl.ds(wid * CPW, CPW)], dst_v)
    plsc.subcore_barrier()

    for b in range(3):
        pltpu.async_copy(g_hbm.at[src_v.at[b]], bufs.at[b], sems[b])

    def body(i, carry):
        for b in range(4):
            j = i * 4 + b
            pltpu.make_async_copy(g_hbm.at[src_v.at[j]], bufs.at[b],
                                  sems[b]).wait()

            @pl.when(j + 3 < CPW)
            def _(j=j, b=b):
                nb = (b + 3) % 4
                pltpu.async_copy(g_hbm.at[src_v.at[j + 3]], bufs.at[nb],
                                 sems[nb])
            pltpu.sync_copy(bufs.at[b], acc_sh.at[dst_v.at[j]], add=True)
        return carry

    lax.fori_loop(0, CPW // 4, body, 0)
    plsc.subcore_barrier()
    pltpu.sync_copy(acc_sh.at[pl.ds(sid * TROWS, TROWS)],
                    out_hbm.at[cid, pl.ds(sid * TROWS, TROWS)])


_BLK = 1000


def _row_spec(width):
    return pl.BlockSpec((_BLK, width), lambda i: (i, 0))


def _part_spec(core, width):
    return pl.BlockSpec((1, _BLK, width), lambda i, c=core: (c, i, 0))


def _full_spec(shape):
    return pl.BlockSpec(shape, lambda i: (0,) * len(shape))


def _tc_prep_body(x_ref, w1_ref, d0_ref, d1_ref, g_ref, dinv_ref):
    deg = d0_ref[0, :, 0:1] + d1_ref[0, :, 0:1] + 1.0
    dinv = lax.rsqrt(deg)
    h = jnp.dot(x_ref[...], w1_ref[...], preferred_element_type=jnp.float32)
    g_ref[...] = h * dinv
    dinv_ref[...] = dinv


def _tc_prep(x, W1, degw):
    return pl.pallas_call(
        _tc_prep_body,
        grid=(N // _BLK,),
        in_specs=[_row_spec(F_IN), _full_spec((F_IN, H)),
                  _part_spec(0, 16), _part_spec(1, 16)],
        out_specs=[_row_spec(H), _row_spec(1)],
        out_shape=[jax.ShapeDtypeStruct((N, H), jnp.float32),
                   jax.ShapeDtypeStruct((N, 1), jnp.float32)],
    )(x, W1, degw, degw)


def _tc_mid_body(p0_ref, p1_ref, g_ref, dinv_ref, b_ref, w_ref, out_ref):
    dinv = dinv_ref[...]
    pre = (p0_ref[0] + p1_ref[0] + g_ref[...]) * dinv + b_ref[...]
    h = jnp.maximum(pre, 0.0)
    out_ref[...] = jnp.dot(h, w_ref[...],
                           preferred_element_type=jnp.float32) * dinv


def _tc_mid(p, g, dinv, b, Wn):
    return pl.pallas_call(
        _tc_mid_body,
        grid=(N // _BLK,),
        in_specs=[_part_spec(0, H), _part_spec(1, H), _row_spec(H),
                  _row_spec(1), _full_spec((1, H)), _full_spec((H, H))],
        out_specs=_row_spec(H),
        out_shape=jax.ShapeDtypeStruct((N, H), jnp.float32),
    )(p, p, g, dinv, b, Wn)


def _tc_final_body(p0_ref, p1_ref, g_ref, dinv_ref, b_ref, wc_ref, bc_ref,
                   wr_ref, br_ref, emb_ref, cls_ref, reg_ref):
    emb = (p0_ref[0] + p1_ref[0] + g_ref[...]) * dinv_ref[...] + b_ref[...]
    emb_ref[...] = emb
    zc = jnp.sum(emb * wc_ref[...], axis=1, keepdims=True) + bc_ref[...]
    cls_ref[...] = jax.nn.sigmoid(zc)
    reg_ref[...] = jnp.sum(emb * wr_ref[...], axis=1, keepdims=True) + br_ref[...]


def _tc_final(p, g, dinv, b3, Wc, bc, Wr, br):
    return pl.pallas_call(
        _tc_final_body,
        grid=(N // _BLK,),
        in_specs=[_part_spec(0, H), _part_spec(1, H), _row_spec(H),
                  _row_spec(1),
                  _full_spec((1, H)), _full_spec((1, H)), _full_spec((1, 1)),
                  _full_spec((1, H)), _full_spec((1, 1))],
        out_specs=[_row_spec(H), _row_spec(1), _row_spec(1)],
        out_shape=[jax.ShapeDtypeStruct((N, H), jnp.float32),
                   jax.ShapeDtypeStruct((N, 1), jnp.float32),
                   jax.ShapeDtypeStruct((N, 1), jnp.float32)],
    )(p, p, g, dinv, b3, Wc, bc, Wr, br)


def kernel(x, edge_index, W1, b1, W2, b2, W3, b3, Wc, bc, Wr, br):
    src = edge_index[0]
    dst = edge_index[1]
    pad = E_PAD - E
    dump = N + jnp.arange(pad, dtype=jnp.int32) % (N_SH - N)
    srcp = jnp.concatenate([src, jnp.zeros((pad,), jnp.int32)])
    dstp = jnp.concatenate([dst, dump])
    srcp = srcp.reshape(NW * CPW, CHUNK)
    dstp = dstp.reshape(NW * CPW, CHUNK)
    ones16 = jnp.ones((CHUNK, 16), jnp.float32)
    zero16 = jnp.zeros((TROWS, 16), jnp.float32)
    zero64 = jnp.zeros((TROWS, H), jnp.float32)

    degw = _sc_degree(dstp, ones16, zero16)
    g1, dinv = _tc_prep(x, W1, degw)
    p = _sc_aggregate(g1, srcp, dstp, zero64)
    g2 = _tc_mid(p, g1, dinv, b1.reshape(1, H), W2)
    p = _sc_aggregate(g2, srcp, dstp, zero64)
    g3 = _tc_mid(p, g2, dinv, b2.reshape(1, H), W3)
    p = _sc_aggregate(g3, srcp, dstp, zero64)
    emb, cls, reg = _tc_final(p, g3, dinv, b3.reshape(1, H),
                              Wc.reshape(1, H), bc.reshape(1, 1),
                              Wr.reshape(1, H), br.reshape(1, 1))
    return (emb, cls, reg)


# speedup vs baseline: 2.2369x; 2.2369x over previous
import functools

import jax
import jax.numpy as jnp
from jax import lax
from jax.experimental import pallas as pl
from jax.experimental.pallas import tpu as pltpu
from jax.experimental.pallas import tpu_sc as plsc

N = 10000
E = 320000
F_IN = 128
H = 64

NC = 2
NS = 16
NW = NC * NS
CHUNK = 128
CPW = 80
E_PAD = NW * CPW * CHUNK
TROWS = 632
N_SH = NS * TROWS

_mesh = plsc.VectorSubcoreMesh(core_axis_name="c", subcore_axis_name="s")
_sc_params = pltpu.CompilerParams(use_tc_tiling_on_sc=False)


@functools.partial(
    pl.kernel,
    out_type=jax.ShapeDtypeStruct((NC, N_SH, 16), jnp.float32),
    mesh=_mesh,
    scratch_types=[
        pltpu.VMEM((CPW, CHUNK), jnp.int32),
        pltpu.VMEM((CHUNK, 16), jnp.float32),
        pltpu.VMEM_SHARED((N_SH, 16), jnp.float32),
    ],
    compiler_params=_sc_params,
)
def _sc_degree(dst_hbm, ones_hbm, zero_hbm, out_hbm, dst_v, ones_v, acc_sh):
    cid = lax.axis_index("c")
    sid = lax.axis_index("s")
    wid = sid * NC + cid
    pltpu.sync_copy(zero_hbm, acc_sh.at[pl.ds(sid * TROWS, TROWS)])
    pltpu.sync_copy(dst_hbm.at[pl.ds(wid * CPW, CPW)], dst_v)
    pltpu.sync_copy(ones_hbm, ones_v)
    plsc.subcore_barrier()

    def body(j, carry):
        pltpu.sync_copy(ones_v, acc_sh.at[dst_v.at[j]], add=True)
        return carry

    lax.fori_loop(0, CPW, body, 0, unroll=4)
    plsc.subcore_barrier()
    pltpu.sync_copy(acc_sh.at[pl.ds(sid * TROWS, TROWS)],
                    out_hbm.at[cid, pl.ds(sid * TROWS, TROWS)])


@functools.partial(
    pl.kernel,
    out_type=jax.ShapeDtypeStruct((NC, N_SH, H), jnp.float32),
    mesh=_mesh,
    scratch_types=[
        pltpu.VMEM((CPW, CHUNK), jnp.int32),
        pltpu.VMEM((CPW, CHUNK), jnp.int32),
        pltpu.VMEM((4, CHUNK, H // 2), jnp.float32),
        pltpu.VMEM_SHARED((N_SH, H // 2), jnp.float32),
        pltpu.VMEM_SHARED((N_SH, H // 2), jnp.float32),
        pltpu.SemaphoreType.DMA,
        pltpu.SemaphoreType.DMA,
        pltpu.SemaphoreType.DMA,
        pltpu.SemaphoreType.DMA,
    ],
    compiler_params=_sc_params,
)
def _sc_aggregate(g_hbm, src_hbm, dst_hbm, zero_hbm, out_hbm,
                  src_v, dst_v, bufs, g_sh, acc_sh, s0, s1, s2, s3):
    cid = lax.axis_index("c")
    sid = lax.axis_index("s")
    wid = sid * NC + cid
    sems = (s0, s1, s2, s3)
    HH = H // 2
    pltpu.sync_copy(src_hbm.at[pl.ds(wid * CPW, CPW)], src_v)
    pltpu.sync_copy(dst_hbm.at[pl.ds(wid * CPW, CPW)], dst_v)

    for phase in range(2):
        pltpu.sync_copy(zero_hbm, acc_sh.at[pl.ds(sid * TROWS, TROWS)])
        pltpu.sync_copy(
            g_hbm.at[pl.ds(sid * (N // NS), N // NS),
                     pl.ds(phase * HH, HH)],
            g_sh.at[pl.ds(sid * (N // NS), N // NS)])
        plsc.subcore_barrier()

        for b in range(3):
            pltpu.async_copy(g_sh.at[src_v.at[b]], bufs.at[b], sems[b])

        def body(i, carry):
            for b in range(4):
                j = i * 4 + b
                pltpu.make_async_copy(g_sh.at[src_v.at[j]], bufs.at[b],
                                      sems[b]).wait()

                @pl.when(j + 3 < CPW)
                def _(j=j, b=b):
                    nb = (b + 3) % 4
                    pltpu.async_copy(g_sh.at[src_v.at[j + 3]], bufs.at[nb],
                                     sems[nb])
                pltpu.sync_copy(bufs.at[b], acc_sh.at[dst_v.at[j]], add=True)
            return carry

        lax.fori_loop(0, CPW // 4, body, 0)
        plsc.subcore_barrier()
        pltpu.sync_copy(acc_sh.at[pl.ds(sid * TROWS, TROWS)],
                        out_hbm.at[cid, pl.ds(sid * TROWS, TROWS),
                                   pl.ds(phase * HH, HH)])


_BLK = 1000


def _row_spec(width):
    return pl.BlockSpec((_BLK, width), lambda i: (i, 0))


def _part_spec(core, width):
    return pl.BlockSpec((1, _BLK, width), lambda i, c=core: (c, i, 0))


def _full_spec(shape):
    return pl.BlockSpec(shape, lambda i: (0,) * len(shape))


def _tc_prep_body(x_ref, w1_ref, d0_ref, d1_ref, g_ref, dinv_ref):
    deg = d0_ref[0, :, 0:1] + d1_ref[0, :, 0:1] + 1.0
    dinv = lax.rsqrt(deg)
    h = jnp.dot(x_ref[...], w1_ref[...], preferred_element_type=jnp.float32)
    g_ref[...] = h * dinv
    dinv_ref[...] = dinv


def _tc_prep(x, W1, degw):
    return pl.pallas_call(
        _tc_prep_body,
        grid=(N // _BLK,),
        in_specs=[_row_spec(F_IN), _full_spec((F_IN, H)),
                  _part_spec(0, 16), _part_spec(1, 16)],
        out_specs=[_row_spec(H), _row_spec(1)],
        out_shape=[jax.ShapeDtypeStruct((N, H), jnp.float32),
                   jax.ShapeDtypeStruct((N, 1), jnp.float32)],
    )(x, W1, degw, degw)


def _tc_mid_body(p0_ref, p1_ref, g_ref, dinv_ref, b_ref, w_ref, out_ref):
    dinv = dinv_ref[...]
    pre = (p0_ref[0] + p1_ref[0] + g_ref[...]) * dinv + b_ref[...]
    h = jnp.maximum(pre, 0.0)
    out_ref[...] = jnp.dot(h, w_ref[...],
                           preferred_element_type=jnp.float32) * dinv


def _tc_mid(p, g, dinv, b, Wn):
    return pl.pallas_call(
        _tc_mid_body,
        grid=(N // _BLK,),
        in_specs=[_part_spec(0, H), _part_spec(1, H), _row_spec(H),
                  _row_spec(1), _full_spec((1, H)), _full_spec((H, H))],
        out_specs=_row_spec(H),
        out_shape=jax.ShapeDtypeStruct((N, H), jnp.float32),
    )(p, p, g, dinv, b, Wn)


def _tc_final_body(p0_ref, p1_ref, g_ref, dinv_ref, b_ref, wc_ref, bc_ref,
                   wr_ref, br_ref, emb_ref, cls_ref, reg_ref):
    emb = (p0_ref[0] + p1_ref[0] + g_ref[...]) * dinv_ref[...] + b_ref[...]
    emb_ref[...] = emb
    zc = jnp.sum(emb * wc_ref[...], axis=1, keepdims=True) + bc_ref[...]
    cls_ref[...] = jax.nn.sigmoid(zc)
    reg_ref[...] = jnp.sum(emb * wr_ref[...], axis=1, keepdims=True) + br_ref[...]


def _tc_final(p, g, dinv, b3, Wc, bc, Wr, br):
    return pl.pallas_call(
        _tc_final_body,
        grid=(N // _BLK,),
        in_specs=[_part_spec(0, H), _part_spec(1, H), _row_spec(H),
                  _row_spec(1),
                  _full_spec((1, H)), _full_spec((1, H)), _full_spec((1, 1)),
                  _full_spec((1, H)), _full_spec((1, 1))],
        out_specs=[_row_spec(H), _row_spec(1), _row_spec(1)],
        out_shape=[jax.ShapeDtypeStruct((N, H), jnp.float32),
                   jax.ShapeDtypeStruct((N, 1), jnp.float32),
                   jax.ShapeDtypeStruct((N, 1), jnp.float32)],
    )(p, p, g, dinv, b3, Wc, bc, Wr, br)


def kernel(x, edge_index, W1, b1, W2, b2, W3, b3, Wc, bc, Wr, br):
    src = edge_index[0]
    dst = edge_index[1]
    pad = E_PAD - E
    dump = N + jnp.arange(pad, dtype=jnp.int32) % (N_SH - N)
    srcp = jnp.concatenate([src, jnp.zeros((pad,), jnp.int32)])
    dstp = jnp.concatenate([dst, dump])
    srcp = srcp.reshape(NW * CPW, CHUNK)
    dstp = dstp.reshape(NW * CPW, CHUNK)
    ones16 = jnp.ones((CHUNK, 16), jnp.float32)
    zero16 = jnp.zeros((TROWS, 16), jnp.float32)
    zero64 = jnp.zeros((TROWS, H // 2), jnp.float32)

    degw = _sc_degree(dstp, ones16, zero16)
    g1, dinv = _tc_prep(x, W1, degw)
    p = _sc_aggregate(g1, srcp, dstp, zero64)
    g2 = _tc_mid(p, g1, dinv, b1.reshape(1, H), W2)
    p = _sc_aggregate(g2, srcp, dstp, zero64)
    g3 = _tc_mid(p, g2, dinv, b2.reshape(1, H), W3)
    p = _sc_aggregate(g3, srcp, dstp, zero64)
    emb, cls, reg = _tc_final(p, g3, dinv, b3.reshape(1, H),
                              Wc.reshape(1, H), bc.reshape(1, 1),
                              Wr.reshape(1, H), br.reshape(1, 1))
    return (emb, cls, reg)


# speedup vs baseline: 2.6042x; 1.1642x over previous
import functools

import jax
import jax.numpy as jnp
from jax import lax
from jax.experimental import pallas as pl
from jax.experimental.pallas import tpu as pltpu
from jax.experimental.pallas import tpu_sc as plsc

N = 10000
E = 320000
F_IN = 128
H = 64

NC = 2
NS = 16
NW = NC * NS
CHUNK = 128
CPW = 80
E_CHUNKS = E // CHUNK
TAIL = E_CHUNKS - (NW - 1) * CPW
TROWS = 632
N_SH = NS * TROWS

_mesh = plsc.VectorSubcoreMesh(core_axis_name="c", subcore_axis_name="s")
_sc_params = pltpu.CompilerParams(use_tc_tiling_on_sc=False)


@functools.partial(
    pl.kernel,
    out_type=jax.ShapeDtypeStruct((NC, N_SH, 16), jnp.float32),
    mesh=_mesh,
    scratch_types=[
        pltpu.VMEM((CPW, CHUNK), jnp.int32),
        pltpu.VMEM((CHUNK, 16), jnp.float32),
        pltpu.VMEM_SHARED((N_SH, 16), jnp.float32),
    ],
    compiler_params=_sc_params,
)
def _sc_degree(eidx_hbm, ones_hbm, zero_hbm, out_hbm, dst_v, ones_v, acc_sh):
    cid = lax.axis_index("c")
    sid = lax.axis_index("s")
    wid = sid * NC + cid
    limit = jnp.minimum(E_CHUNKS - wid * CPW, CPW)
    pltpu.sync_copy(zero_hbm, acc_sh.at[pl.ds(sid * TROWS, TROWS)])
    @pl.when(wid < NW - 1)
    def _():
        pltpu.sync_copy(eidx_hbm.at[1, pl.ds(wid * CPW, CPW)], dst_v)

    @pl.when(wid == NW - 1)
    def _():
        pltpu.sync_copy(eidx_hbm.at[1, pl.ds((NW - 1) * CPW, TAIL)],
                        dst_v.at[pl.ds(0, TAIL)])

    pltpu.sync_copy(ones_hbm, ones_v)
    plsc.subcore_barrier()

    def body(j, carry):
        @pl.when(j < limit)
        def _():
            pltpu.sync_copy(ones_v, acc_sh.at[dst_v.at[j]], add=True)
        return carry

    lax.fori_loop(0, CPW, body, 0, unroll=4)
    plsc.subcore_barrier()
    pltpu.sync_copy(acc_sh.at[pl.ds(sid * TROWS, TROWS)],
                    out_hbm.at[cid, pl.ds(sid * TROWS, TROWS)])


@functools.partial(
    pl.kernel,
    out_type=jax.ShapeDtypeStruct((NC, N_SH, 128), jnp.float32),
    mesh=_mesh,
    scratch_types=[
        pltpu.VMEM((CPW, CHUNK), jnp.int32),
        pltpu.VMEM((CPW, CHUNK), jnp.int32),
        pltpu.VMEM((8, CHUNK, H // 2), jnp.float32),
        pltpu.VMEM_SHARED((N_SH, H // 2), jnp.float32),
        pltpu.VMEM_SHARED((N_SH, H // 2), jnp.float32),
    ] + [pltpu.SemaphoreType.DMA] * 16,
    compiler_params=_sc_params,
)
def _sc_aggregate(g_hbm, eidx_hbm, zero_hbm, out_hbm,
                  src_v, dst_v, bufs, g_sh, acc_sh, *sems):
    cid = lax.axis_index("c")
    sid = lax.axis_index("s")
    wid = sid * NC + cid
    limit = jnp.minimum(E_CHUNKS - wid * CPW, CPW)
    gsem = sems[:8]
    ssem = sems[8:]
    HH = H // 2

    @pl.when(wid < NW - 1)
    def _():
        pltpu.sync_copy(eidx_hbm.at[0, pl.ds(wid * CPW, CPW)], src_v)
        pltpu.sync_copy(eidx_hbm.at[1, pl.ds(wid * CPW, CPW)], dst_v)

    @pl.when(wid == NW - 1)
    def _():
        pltpu.sync_copy(eidx_hbm.at[0, pl.ds((NW - 1) * CPW, TAIL)],
                        src_v.at[pl.ds(0, TAIL)])
        pltpu.sync_copy(eidx_hbm.at[1, pl.ds((NW - 1) * CPW, TAIL)],
                        dst_v.at[pl.ds(0, TAIL)])

    def gather(j, b):
        return pltpu.make_async_copy(g_sh.at[src_v.at[j]], bufs.at[b],
                                     gsem[b])

    def scatter(j, b):
        return pltpu.make_async_copy(bufs.at[b], acc_sh.at[dst_v.at[j]],
                                     ssem[b])

    for phase in range(2):
        pltpu.sync_copy(zero_hbm, acc_sh.at[pl.ds(sid * TROWS, TROWS)])
        pltpu.sync_copy(
            g_hbm.at[pl.ds(sid * (N // NS), N // NS),
                     pl.ds(phase * HH, HH)],
            g_sh.at[pl.ds(sid * (N // NS), N // NS)])
        plsc.subcore_barrier()

        for b in range(4):
            gather(b, b).start()

        def body(i, carry):
            for b in range(8):
                j = i * 8 + b
                nb = (b + 4) % 8

                @pl.when(j + 4 < limit)
                def _(j=j, nb=nb):
                    gather(j + 4, nb).start()

                @pl.when(j < limit)
                def _(j=j, b=b):
                    gather(j, b).wait()
                    pltpu.sync_copy(bufs.at[b], acc_sh.at[dst_v.at[j]],
                                    add=True)
            return carry

        lax.fori_loop(0, CPW // 8, body, 0)
        plsc.subcore_barrier()
        pltpu.sync_copy(acc_sh.at[pl.ds(sid * TROWS, TROWS)],
                        out_hbm.at[cid, pl.ds(sid * TROWS, TROWS),
                                   pl.ds(phase * HH, HH)])


_BLK = 1000


def _row_spec(width):
    return pl.BlockSpec((_BLK, width), lambda i: (i, 0))


def _part_spec(core, width):
    return pl.BlockSpec((1, _BLK, width), lambda i, c=core: (c, i, 0))


def _full_spec(shape):
    return pl.BlockSpec(shape, lambda i: (0,) * len(shape))


def _tc_prep_body(x_ref, w1_ref, d0_ref, d1_ref, g_ref, dinv_ref):
    deg = d0_ref[0, :, 0:1] + d1_ref[0, :, 0:1] + 1.0
    dinv = lax.rsqrt(deg)
    h = jnp.dot(x_ref[...], w1_ref[...], preferred_element_type=jnp.float32)
    g = h * dinv
    g_ref[...] = jnp.concatenate([g, g], axis=1)
    dinv_ref[...] = dinv


def _tc_prep(x, W1, degw):
    return pl.pallas_call(
        _tc_prep_body,
        grid=(N // _BLK,),
        in_specs=[_row_spec(F_IN), _full_spec((F_IN, H)),
                  _part_spec(0, 16), _part_spec(1, 16)],
        out_specs=[_row_spec(128), _row_spec(1)],
        out_shape=[jax.ShapeDtypeStruct((N, 128), jnp.float32),
                   jax.ShapeDtypeStruct((N, 1), jnp.float32)],
    )(x, W1, degw, degw)


def _tc_mid_body(p0_ref, p1_ref, g_ref, dinv_ref, b_ref, w_ref, out_ref):
    dinv = dinv_ref[...]
    pre = ((p0_ref[0, :, 0:H] + p1_ref[0, :, 0:H] + g_ref[:, 0:H]) * dinv
           + b_ref[...])
    h = jnp.maximum(pre, 0.0)
    g = jnp.dot(h, w_ref[...], preferred_element_type=jnp.float32) * dinv
    out_ref[...] = jnp.concatenate([g, g], axis=1)


def _tc_mid(p, g, dinv, b, Wn):
    return pl.pallas_call(
        _tc_mid_body,
        grid=(N // _BLK,),
        in_specs=[_part_spec(0, 128), _part_spec(1, 128), _row_spec(128),
                  _row_spec(1), _full_spec((1, H)), _full_spec((H, H))],
        out_specs=_row_spec(128),
        out_shape=jax.ShapeDtypeStruct((N, 128), jnp.float32),
    )(p, p, g, dinv, b, Wn)


def _tc_final_body(p0_ref, p1_ref, g_ref, dinv_ref, b_ref, wc_ref, bc_ref,
                   wr_ref, br_ref, emb_ref, cls_ref, reg_ref):
    emb = ((p0_ref[0, :, 0:H] + p1_ref[0, :, 0:H] + g_ref[:, 0:H])
           * dinv_ref[...] + b_ref[...])
    emb_ref[...] = emb
    zc = jnp.sum(emb * wc_ref[...], axis=1, keepdims=True) + bc_ref[...]
    cls_ref[...] = jax.nn.sigmoid(zc)
    reg_ref[...] = jnp.sum(emb * wr_ref[...], axis=1, keepdims=True) + br_ref[...]


def _tc_final(p, g, dinv, b3, Wc, bc, Wr, br):
    return pl.pallas_call(
        _tc_final_body,
        grid=(N // _BLK,),
        in_specs=[_part_spec(0, 128), _part_spec(1, 128), _row_spec(128),
                  _row_spec(1),
                  _full_spec((1, H)), _full_spec((1, H)), _full_spec((1, 1)),
                  _full_spec((1, H)), _full_spec((1, 1))],
        out_specs=[_row_spec(H), _row_spec(1), _row_spec(1)],
        out_shape=[jax.ShapeDtypeStruct((N, H), jnp.float32),
                   jax.ShapeDtypeStruct((N, 1), jnp.float32),
                   jax.ShapeDtypeStruct((N, 1), jnp.float32)],
    )(p, p, g, dinv, b3, Wc, bc, Wr, br)


def kernel(x, edge_index, W1, b1, W2, b2, W3, b3, Wc, bc, Wr, br):
    epad = edge_index.reshape(2, E_CHUNKS, CHUNK)
    ones16 = jnp.ones((CHUNK, 16), jnp.float32)
    zero16 = jnp.zeros((TROWS, 16), jnp.float32)
    zero64 = jnp.zeros((TROWS, H // 2), jnp.float32)

    degw = _sc_degree(epad, ones16, zero16)
    g1, dinv = _tc_prep(x, W1, degw)
    p = _sc_aggregate(g1, epad, zero64)
    g2 = _tc_mid(p, g1, dinv, b1.reshape(1, H), W2)
    p = _sc_aggregate(g2, epad, zero64)
    g3 = _tc_mid(p, g2, dinv, b2.reshape(1, H), W3)
    p = _sc_aggregate(g3, epad, zero64)
    emb, cls, reg = _tc_final(p, g3, dinv, b3.reshape(1, H),
                              Wc.reshape(1, H), bc.reshape(1, 1),
                              Wr.reshape(1, H), br.reshape(1, 1))
    return (emb, cls, reg)
